# Initial kernel scaffold; baseline (speedup 1.0000x reference)
import functools

import jax
import jax.numpy as jnp
from jax import lax
from jax.experimental import pallas as pl
from jax.experimental.pallas import tpu as pltpu
from jax.experimental.pallas import tpu_sc as plsc

_N = 10000
_E = 320000
_H = 128
_NC = 2
_NS = 16
_NW = _NC * _NS
_CH = 128
_N_PAD = 12288
_E_PAD = 323584
_BE = 2048
_BN = 2048

_f32 = jnp.float32


def _mesh():
    return plsc.VectorSubcoreMesh(
        core_axis_name="c", subcore_axis_name="s",
        num_cores=_NC, num_subcores=_NS)


def _sc_gather(table, idx3d, width):
    nw, k, _ = idx3d.shape

    @functools.partial(
        pl.kernel,
        out_type=jax.ShapeDtypeStruct((nw * k * _CH, width), _f32),
        mesh=_mesh(),
        scratch_types=[
            pltpu.VMEM((k, _CH), jnp.int32),
            pltpu.VMEM((_CH, width), _f32),
            pltpu.SemaphoreType.DMA,
        ],
    )
    def gather_k(table_hbm, idx_hbm, out_hbm, idx_v, rows_v, sem):
        c = lax.axis_index("c")
        s = lax.axis_index("s")
        w = s * _NC + c
        pltpu.sync_copy(idx_hbm.at[w], idx_v)
        base = w * (k * _CH)

        def body(j, carry):
            pltpu.async_copy(table_hbm.at[idx_v.at[j]], rows_v, sem).wait()
            pltpu.sync_copy(rows_v, out_hbm.at[pl.ds(base + j * _CH, _CH)])
            return carry

        lax.fori_loop(0, k, body, 0)

    return gather_k(table, idx3d)


def _sc_scatter_add(m, dst3d, zrows):
    nw, k, _ = dst3d.shape
    rows_per_s = _N_PAD // _NS

    @functools.partial(
        pl.kernel,
        out_type=jax.ShapeDtypeStruct((_NC, _N_PAD, _H), _f32),
        mesh=_mesh(),
        scratch_types=[
            pltpu.VMEM((k, _CH), jnp.int32),
            pltpu.VMEM((_CH, _H), _f32),
            pltpu.VMEM_SHARED((_N_PAD, _H), _f32),
        ],
    )
    def scatter_k(m_hbm, dst_hbm, z_hbm, out_hbm, idx_v, mbuf, acc):
        c = lax.axis_index("c")
        s = lax.axis_index("s")
        w = s * _NC + c
        pltpu.sync_copy(z_hbm, acc.at[pl.ds(s * rows_per_s, rows_per_s)])
        plsc.subcore_barrier()
        pltpu.sync_copy(dst_hbm.at[w], idx_v)
        base = w * (k * _CH)

        def body(j, carry):
            pltpu.sync_copy(m_hbm.at[pl.ds(base + j * _CH, _CH)], mbuf)
            pltpu.sync_copy(mbuf, acc.at[idx_v.at[j]], add=True)
            return carry

        lax.fori_loop(0, k, body, 0)
        plsc.subcore_barrier()
        pltpu.sync_copy(acc.at[pl.ds(s * rows_per_s, rows_per_s)],
                        out_hbm.at[c, pl.ds(s * rows_per_s, rows_per_s)])

    return scatter_k(m, dst3d, zrows)


def _ln(h):
    mu = jnp.mean(h, axis=-1, keepdims=True)
    d = h - mu
    var = jnp.mean(d * d, axis=-1, keepdims=True)
    return d * lax.rsqrt(var + 1e-5)


def _dot(a, b):
    return jnp.dot(a, b, preferred_element_type=_f32)


def _full(spec_shape):
    return pl.BlockSpec(spec_shape, lambda i: tuple(0 for _ in spec_shape))


def _mlp3_body(x_ref, w1, b1, w2, b2, w3, b3, o_ref, *, layernorm):
    h = jnp.maximum(_dot(x_ref[...], w1[...]) + b1[...], 0.0)
    h = jnp.maximum(_dot(h, w2[...]) + b2[...], 0.0)
    h = _dot(h, w3[...]) + b3[...]
    o_ref[...] = _ln(h) if layernorm else h


def _mlp3(x, params, *, layernorm, block, out_dim):
    n, din = x.shape
    (w1, b1), (w2, b2), (w3, b3) = params
    grid = (n // block,)
    return pl.pallas_call(
        functools.partial(_mlp3_body, layernorm=layernorm),
        grid=grid,
        in_specs=[
            pl.BlockSpec((block, din), lambda i: (i, 0)),
            _full(w1.shape), _full((1, w1.shape[1])),
            _full(w2.shape), _full((1, w2.shape[1])),
            _full(w3.shape), _full((1, w3.shape[1])),
        ],
        out_specs=pl.BlockSpec((block, out_dim), lambda i: (i, 0)),
        out_shape=jax.ShapeDtypeStruct((n, out_dim), _f32),
    )(x, w1, b1.reshape(1, -1), w2, b2.reshape(1, -1), w3, b3.reshape(1, -1))


def _pmat_body(nf_ref, wa, wb, o_ref):
    nf = nf_ref[...]
    o_ref[0] = _dot(nf, wa[...])
    o_ref[1] = _dot(nf, wb[...])


def _pmat(nf, wa, wb):
    grid = (_N_PAD // _BN,)
    return pl.pallas_call(
        _pmat_body,
        grid=grid,
        in_specs=[
            pl.BlockSpec((_BN, _H), lambda i: (i, 0)),
            _full((_H, _H)), _full((_H, _H)),
        ],
        out_specs=pl.BlockSpec((2, _BN, _H), lambda i: (0, i, 0)),
        out_shape=jax.ShapeDtypeStruct((2, _N_PAD, _H), _f32),
    )(nf, wa, wb)


def _edge_body(ga_ref, gb_ref, ef_ref, nd_ref, w1c, b1, w2, b2, w3, b3,
               wd1, bd1, wd2, bd2, m_ref, efo_ref):
    ef = ef_ref[...]
    h = ga_ref[0] + gb_ref[0] + _dot(ef, w1c[...]) + b1[...]
    h = jnp.maximum(h, 0.0)
    h = jnp.maximum(_dot(h, w2[...]) + b2[...], 0.0)
    h = _dot(h, w3[...]) + b3[...]
    m0 = _ln(h)
    nd = nd_ref[0]
    hd = jnp.maximum(nd * wd1[...] + bd1[...], 0.0)
    wgt = _dot(hd, wd2[...]) + bd2[...]
    m = m0 * wgt
    m_ref[...] = m
    efo_ref[...] = ef + m


def _edge_mlp(g, ef, nd3, p_edge, p_dist):
    (w1, b1), (w2, b2), (w3, b3) = p_edge
    (wd1, bd1), (wd2, bd2) = p_dist
    w1c = w1[2 * _H:]
    grid = (_E_PAD // _BE,)
    return pl.pallas_call(
        _edge_body,
        grid=grid,
        in_specs=[
            pl.BlockSpec((1, _BE, _H), lambda i: (0, i, 0)),
            pl.BlockSpec((1, _BE, _H), lambda i: (1, i, 0)),
            pl.BlockSpec((_BE, _H), lambda i: (i, 0)),
            pl.BlockSpec((1, _BE, 1), lambda i: (i, 0, 0)),
            _full((_H, _H)), _full((1, _H)),
            _full((_H, _H)), _full((1, _H)),
            _full((_H, _H)), _full((1, _H)),
            _full((1, _H)), _full((1, _H)),
            _full((_H, _H)), _full((1, _H)),
        ],
        out_specs=[
            pl.BlockSpec((_BE, _H), lambda i: (i, 0)),
            pl.BlockSpec((_BE, _H), lambda i: (i, 0)),
        ],
        out_shape=[
            jax.ShapeDtypeStruct((_E_PAD, _H), _f32),
            jax.ShapeDtypeStruct((_E_PAD, _H), _f32),
        ],
    )(g, g, ef, nd3, w1c, b1.reshape(1, -1), w2, b2.reshape(1, -1),
      w3, b3.reshape(1, -1), wd1, bd1.reshape(1, -1), wd2, bd2.reshape(1, -1))


def _node_body(nf_ref, p0_ref, p1_ref, wa, wb, b1, w2, b2, w3, b3, o_ref):
    nf = nf_ref[...]
    aggr = p0_ref[0] + p1_ref[0]
    h = jnp.maximum(_dot(nf, wa[...]) + _dot(aggr, wb[...]) + b1[...], 0.0)
    h = jnp.maximum(_dot(h, w2[...]) + b2[...], 0.0)
    h = _dot(h, w3[...]) + b3[...]
    o_ref[...] = nf + _ln(h)


def _node_update(nf, partials, p_node):
    (w1, b1), (w2, b2), (w3, b3) = p_node
    wa, wb = w1[:_H], w1[_H:]
    grid = (_N_PAD // _BN,)
    return pl.pallas_call(
        _node_body,
        grid=grid,
        in_specs=[
            pl.BlockSpec((_BN, _H), lambda i: (i, 0)),
            pl.BlockSpec((1, _BN, _H), lambda i: (0, i, 0)),
            pl.BlockSpec((1, _BN, _H), lambda i: (1, i, 0)),
            _full((_H, _H)), _full((_H, _H)), _full((1, _H)),
            _full((_H, _H)), _full((1, _H)),
            _full((_H, _H)), _full((1, _H)),
        ],
        out_specs=pl.BlockSpec((_BN, _H), lambda i: (i, 0)),
        out_shape=jax.ShapeDtypeStruct((_N_PAD, _H), _f32),
    )(nf, partials, partials, wa, wb, b1.reshape(1, -1), w2, b2.reshape(1, -1),
      w3, b3.reshape(1, -1))


def kernel(x, pos, edge_index, edge_attr, node_dist, params):
    src = edge_index[0].astype(jnp.int32)
    dst = edge_index[1].astype(jnp.int32)
    ep = _E_PAD - _E
    np_ = _N_PAD - _N

    dst_g = jnp.concatenate([dst, jnp.zeros((ep,), jnp.int32)])
    src_g = jnp.concatenate([src, jnp.zeros((ep,), jnp.int32)])
    idx_comb = jnp.concatenate([dst_g, src_g + _N_PAD]).reshape(_NW, -1, _CH)
    dst_s = jnp.concatenate(
        [dst, jnp.full((ep,), _N, jnp.int32)]).reshape(_NW, -1, _CH)
    x_pad = jnp.concatenate(
        [x.astype(jnp.int32), jnp.zeros((np_,), jnp.int32)]).reshape(_NW, -1, _CH)
    pos_pad = jnp.pad(pos.astype(_f32), ((0, np_), (0, 0)))
    ea_pad = jnp.pad(edge_attr.astype(_f32), ((0, ep), (0, 0)))
    nd3 = jnp.pad(node_dist.astype(_f32), (0, ep)).reshape(
        _E_PAD // _BE, _BE, 1)
    zrows = jnp.zeros((_N_PAD // _NS, _H), _f32)

    emb = _sc_gather(params['embed'].astype(_f32), x_pad, 16)
    xc = jnp.concatenate([emb, pos_pad], axis=1)
    nf = _mlp3(xc, params['node_in'], layernorm=True, block=_BN, out_dim=_H)
    ef = _mlp3(ea_pad, params['edge_in'], layernorm=True, block=_BE, out_dim=_H)

    for p in params['layers']:
        w1 = p['edge_mlp'][0][0]
        pm = _pmat(nf, w1[:_H], w1[_H:2 * _H])
        g = _sc_gather(pm.reshape(2 * _N_PAD, _H), idx_comb, _H)
        g = g.reshape(2, _E_PAD, _H)
        m, ef = _edge_mlp(g, ef, nd3, p['edge_mlp'], p['dist'])
        partials = _sc_scatter_add(m, dst_s, zrows)
        nf = _node_update(nf, partials, p['node_mlp'])

    out = _mlp3(nf, params['node_out'], layernorm=False, block=_BN, out_dim=3)
    return out[:_N]


# speedup vs baseline: 2.4936x; 2.4936x over previous
import functools

import jax
import jax.numpy as jnp
from jax import lax
from jax.experimental import pallas as pl
from jax.experimental.pallas import tpu as pltpu
from jax.experimental.pallas import tpu_sc as plsc

_N = 10000
_E = 320000
_H = 128
_NC = 2
_NS = 16
_NW = _NC * _NS
_CH = 128
_N_PAD = 12288
_E_PAD = 323584
_BE = 2048
_BN = 2048

_f32 = jnp.float32


def _mesh():
    return plsc.VectorSubcoreMesh(
        core_axis_name="c", subcore_axis_name="s",
        num_cores=_NC, num_subcores=_NS)


def _sc_gather(table, idx3d, width):
    nw, k, _ = idx3d.shape

    @functools.partial(
        pl.kernel,
        out_type=jax.ShapeDtypeStruct((nw * k * _CH, width), _f32),
        mesh=_mesh(),
        scratch_types=[
            pltpu.VMEM((k, _CH), jnp.int32),
            pltpu.VMEM((_CH, width), _f32),
            pltpu.SemaphoreType.DMA,
        ],
    )
    def gather_k(table_hbm, idx_hbm, out_hbm, idx_v, rows_v, sem):
        c = lax.axis_index("c")
        s = lax.axis_index("s")
        w = s * _NC + c
        pltpu.sync_copy(idx_hbm.at[w], idx_v)
        base = w * (k * _CH)

        def body(j, carry):
            pltpu.async_copy(table_hbm.at[idx_v.at[j]], rows_v, sem).wait()
            pltpu.sync_copy(rows_v, out_hbm.at[pl.ds(base + j * _CH, _CH)])
            return carry

        lax.fori_loop(0, k, body, 0)

    return gather_k(table, idx3d)


def _sc_scatter_add(m, dst3d, zrows):
    nw, k, _ = dst3d.shape
    rows_per_s = _N_PAD // _NS

    @functools.partial(
        pl.kernel,
        out_type=jax.ShapeDtypeStruct((_NC, _N_PAD, _H), _f32),
        mesh=_mesh(),
        scratch_types=[
            pltpu.VMEM((k, _CH), jnp.int32),
            pltpu.VMEM((_CH, _H), _f32),
            pltpu.VMEM_SHARED((_N_PAD, _H), _f32),
        ],
    )
    def scatter_k(m_hbm, dst_hbm, z_hbm, out_hbm, idx_v, mbuf, acc):
        c = lax.axis_index("c")
        s = lax.axis_index("s")
        w = s * _NC + c
        pltpu.sync_copy(z_hbm, acc.at[pl.ds(s * rows_per_s, rows_per_s)])
        plsc.subcore_barrier()
        pltpu.sync_copy(dst_hbm.at[w], idx_v)
        base = w * (k * _CH)

        def body(j, carry):
            pltpu.sync_copy(m_hbm.at[pl.ds(base + j * _CH, _CH)], mbuf)
            pltpu.sync_copy(mbuf, acc.at[idx_v.at[j]], add=True)
            return carry

        lax.fori_loop(0, k, body, 0)
        plsc.subcore_barrier()
        pltpu.sync_copy(acc.at[pl.ds(s * rows_per_s, rows_per_s)],
                        out_hbm.at[c, pl.ds(s * rows_per_s, rows_per_s)])

    return scatter_k(m, dst3d, zrows)


def _ln(h):
    mu = jnp.mean(h, axis=-1, keepdims=True)
    d = h - mu
    var = jnp.mean(d * d, axis=-1, keepdims=True)
    return d * lax.rsqrt(var + 1e-5)


def _dot(a, b):
    return jnp.dot(a, b, preferred_element_type=_f32)


def _full(spec_shape):
    return pl.BlockSpec(spec_shape, lambda i: tuple(0 for _ in spec_shape))


def _mlp3_body(x_ref, w1, b1, w2, b2, w3, b3, o_ref, *, layernorm):
    h = jnp.maximum(_dot(x_ref[...], w1[...]) + b1[...], 0.0)
    h = jnp.maximum(_dot(h, w2[...]) + b2[...], 0.0)
    h = _dot(h, w3[...]) + b3[...]
    o_ref[...] = _ln(h) if layernorm else h


def _mlp3(x, params, *, layernorm, block, out_dim):
    n, din = x.shape
    (w1, b1), (w2, b2), (w3, b3) = params
    grid = (n // block,)
    return pl.pallas_call(
        functools.partial(_mlp3_body, layernorm=layernorm),
        grid=grid,
        in_specs=[
            pl.BlockSpec((block, din), lambda i: (i, 0)),
            _full(w1.shape), _full((1, w1.shape[1])),
            _full(w2.shape), _full((1, w2.shape[1])),
            _full(w3.shape), _full((1, w3.shape[1])),
        ],
        out_specs=pl.BlockSpec((block, out_dim), lambda i: (i, 0)),
        out_shape=jax.ShapeDtypeStruct((n, out_dim), _f32),
    )(x, w1, b1.reshape(1, -1), w2, b2.reshape(1, -1), w3, b3.reshape(1, -1))


def _embt_body(e_ref, w_ref, o_ref):
    o_ref[...] = _dot(e_ref[...], w_ref[...])


def _emb_table(embed_pad, w1a):
    return pl.pallas_call(
        _embt_body,
        grid=(1,),
        in_specs=[_full((16, 16)), _full((16, _H))],
        out_specs=pl.BlockSpec((16, _H), lambda i: (0, 0)),
        out_shape=jax.ShapeDtypeStruct((16, _H), _f32),
    )(embed_pad, w1a)


def _nodein_body(g_ref, pos_ref, w1p, b1, w2, b2, w3, b3, o_ref):
    h = jnp.maximum(g_ref[...] + _dot(pos_ref[...], w1p[...]) + b1[...], 0.0)
    h = jnp.maximum(_dot(h, w2[...]) + b2[...], 0.0)
    h = _dot(h, w3[...]) + b3[...]
    o_ref[...] = _ln(h)


def _node_in(g_emb, pos_pad, params):
    (w1, b1), (w2, b2), (w3, b3) = params
    w1p = w1[16:]
    dp = w1p.shape[0]
    grid = (_N_PAD // _BN,)
    return pl.pallas_call(
        _nodein_body,
        grid=grid,
        in_specs=[
            pl.BlockSpec((_BN, _H), lambda i: (i, 0)),
            pl.BlockSpec((_BN, dp), lambda i: (i, 0)),
            _full((dp, _H)), _full((1, _H)),
            _full((_H, _H)), _full((1, _H)),
            _full((_H, _H)), _full((1, _H)),
        ],
        out_specs=pl.BlockSpec((_BN, _H), lambda i: (i, 0)),
        out_shape=jax.ShapeDtypeStruct((_N_PAD, _H), _f32),
    )(g_emb, pos_pad, w1p, b1.reshape(1, -1), w2, b2.reshape(1, -1),
      w3, b3.reshape(1, -1))


def _pmat_body(nf_ref, wa, wb, o_ref):
    nf = nf_ref[...]
    o_ref[0] = _dot(nf, wa[...])
    o_ref[1] = _dot(nf, wb[...])


def _pmat(nf, wa, wb):
    grid = (_N_PAD // _BN,)
    return pl.pallas_call(
        _pmat_body,
        grid=grid,
        in_specs=[
            pl.BlockSpec((_BN, _H), lambda i: (i, 0)),
            _full((_H, _H)), _full((_H, _H)),
        ],
        out_specs=pl.BlockSpec((2, _BN, _H), lambda i: (0, i, 0)),
        out_shape=jax.ShapeDtypeStruct((2, _N_PAD, _H), _f32),
    )(nf, wa, wb)


def _edge_body(ga_ref, gb_ref, ef_ref, nd_ref, w1c, b1, w2, b2, w3, b3,
               wd1, bd1, wd2, bd2, m_ref, efo_ref):
    ef = ef_ref[...]
    h = ga_ref[0] + gb_ref[0] + _dot(ef, w1c[...]) + b1[...]
    h = jnp.maximum(h, 0.0)
    h = jnp.maximum(_dot(h, w2[...]) + b2[...], 0.0)
    h = _dot(h, w3[...]) + b3[...]
    m0 = _ln(h)
    nd = nd_ref[0]
    hd = jnp.maximum(nd * wd1[...] + bd1[...], 0.0)
    wgt = _dot(hd, wd2[...]) + bd2[...]
    m = m0 * wgt
    m_ref[...] = m
    efo_ref[...] = ef + m


def _edge_mlp(g, ef, nd3, p_edge, p_dist):
    (w1, b1), (w2, b2), (w3, b3) = p_edge
    (wd1, bd1), (wd2, bd2) = p_dist
    w1c = w1[2 * _H:]
    grid = (_E_PAD // _BE,)
    return pl.pallas_call(
        _edge_body,
        grid=grid,
        in_specs=[
            pl.BlockSpec((1, _BE, _H), lambda i: (0, i, 0)),
            pl.BlockSpec((1, _BE, _H), lambda i: (1, i, 0)),
            pl.BlockSpec((_BE, _H), lambda i: (i, 0)),
            pl.BlockSpec((1, _BE, 1), lambda i: (i, 0, 0)),
            _full((_H, _H)), _full((1, _H)),
            _full((_H, _H)), _full((1, _H)),
            _full((_H, _H)), _full((1, _H)),
            _full((1, _H)), _full((1, _H)),
            _full((_H, _H)), _full((1, _H)),
        ],
        out_specs=[
            pl.BlockSpec((_BE, _H), lambda i: (i, 0)),
            pl.BlockSpec((_BE, _H), lambda i: (i, 0)),
        ],
        out_shape=[
            jax.ShapeDtypeStruct((_E_PAD, _H), _f32),
            jax.ShapeDtypeStruct((_E_PAD, _H), _f32),
        ],
    )(g, g, ef, nd3, w1c, b1.reshape(1, -1), w2, b2.reshape(1, -1),
      w3, b3.reshape(1, -1), wd1, bd1.reshape(1, -1), wd2, bd2.reshape(1, -1))


def _node_body(nf_ref, p0_ref, p1_ref, wa, wb, b1, w2, b2, w3, b3, o_ref):
    nf = nf_ref[...]
    aggr = p0_ref[0] + p1_ref[0]
    h = jnp.maximum(_dot(nf, wa[...]) + _dot(aggr, wb[...]) + b1[...], 0.0)
    h = jnp.maximum(_dot(h, w2[...]) + b2[...], 0.0)
    h = _dot(h, w3[...]) + b3[...]
    o_ref[...] = nf + _ln(h)


def _node_update(nf, partials, p_node):
    (w1, b1), (w2, b2), (w3, b3) = p_node
    wa, wb = w1[:_H], w1[_H:]
    grid = (_N_PAD // _BN,)
    return pl.pallas_call(
        _node_body,
        grid=grid,
        in_specs=[
            pl.BlockSpec((_BN, _H), lambda i: (i, 0)),
            pl.BlockSpec((1, _BN, _H), lambda i: (0, i, 0)),
            pl.BlockSpec((1, _BN, _H), lambda i: (1, i, 0)),
            _full((_H, _H)), _full((_H, _H)), _full((1, _H)),
            _full((_H, _H)), _full((1, _H)),
            _full((_H, _H)), _full((1, _H)),
        ],
        out_specs=pl.BlockSpec((_BN, _H), lambda i: (i, 0)),
        out_shape=jax.ShapeDtypeStruct((_N_PAD, _H), _f32),
    )(nf, partials, partials, wa, wb, b1.reshape(1, -1), w2, b2.reshape(1, -1),
      w3, b3.reshape(1, -1))


def kernel(x, pos, edge_index, edge_attr, node_dist, params):
    src = edge_index[0].astype(jnp.int32)
    dst = edge_index[1].astype(jnp.int32)
    ep = _E_PAD - _E
    np_ = _N_PAD - _N

    dst_g = jnp.concatenate([dst, jnp.zeros((ep,), jnp.int32)])
    src_g = jnp.concatenate([src, jnp.zeros((ep,), jnp.int32)])
    idx_comb = jnp.concatenate([dst_g, src_g + _N_PAD]).reshape(_NW, -1, _CH)
    dst_s = jnp.concatenate(
        [dst, jnp.full((ep,), _N, jnp.int32)]).reshape(_NW, -1, _CH)
    x_pad = jnp.concatenate(
        [x.astype(jnp.int32), jnp.zeros((np_,), jnp.int32)]).reshape(_NW, -1, _CH)
    pos_pad = jnp.pad(pos.astype(_f32), ((0, np_), (0, 0)))
    ea_pad = jnp.pad(edge_attr.astype(_f32), ((0, ep), (0, 0)))
    nd3 = jnp.pad(node_dist.astype(_f32), (0, ep)).reshape(
        _E_PAD // _BE, _BE, 1)
    zrows = jnp.zeros((_N_PAD // _NS, _H), _f32)

    embed_pad = jnp.pad(params['embed'].astype(_f32), ((0, 7), (0, 0)))
    w1_in = params['node_in'][0][0]
    embt = _emb_table(embed_pad, w1_in[:16])
    g_emb = _sc_gather(embt, x_pad, _H)
    nf = _node_in(g_emb, pos_pad, params['node_in'])
    ef = _mlp3(ea_pad, params['edge_in'], layernorm=True, block=_BE, out_dim=_H)

    for p in params['layers']:
        w1 = p['edge_mlp'][0][0]
        pm = _pmat(nf, w1[:_H], w1[_H:2 * _H])
        g = _sc_gather(pm.reshape(2 * _N_PAD, _H), idx_comb, _H)
        g = g.reshape(2, _E_PAD, _H)
        m, ef = _edge_mlp(g, ef, nd3, p['edge_mlp'], p['dist'])
        partials = _sc_scatter_add(m, dst_s, zrows)
        nf = _node_update(nf, partials, p['node_mlp'])

    out = _mlp3(nf, params['node_out'], layernorm=False, block=_BN, out_dim=3)
    return out[:_N]
